# baseline (speedup 1.0000x reference)
import functools

import jax
import jax.numpy as jnp
from jax import lax
from jax.experimental import pallas as pl
from jax.experimental.pallas import tpu as pltpu
from jax.experimental.pallas import tpu_sc as plsc

B, C, H, W = 8, 96, 224, 224
HWN = H * W
NPIX = B * HWN
NC, NS, L = 2, 16, 16
NW = NC * NS
PPW = NPIX // NW
QH = 56
NHR = QH // 8
HC = C // 2
TW = 128
SCW = 128
SCG = SCW // L
TCW = 128
TCV = W - SCW


def _sc_body(scores_hbm, y_hbm, out_hbm, y_v, buf0_v, buf1_v, acc_v,
             sem0, sem1):
    c = lax.axis_index("c")
    s = lax.axis_index("s")
    wid = s * NC + c
    b = wid // 4
    h0 = (wid % 4) * QH

    pltpu.sync_copy(
        y_hbm.at[b, pl.ds(h0, QH), pl.ds(0, SCW)], y_v
    )
    lane = lax.iota(jnp.int32, L)

    bufs = (buf0_v, buf1_v)
    sems = (sem0, sem1)

    def start(stage):
        hr, half = divmod(stage, 2)
        habs = h0 + hr * 8
        return pltpu.async_copy(
            scores_hbm.at[b, pl.ds(half * HC, HC), pl.ds(habs, 8),
                          pl.ds(0, TW)],
            bufs[stage % 2],
            sems[stage % 2],
        )

    def make_grp(hr, half, buf):
        lo = half * HC

        def grp(i, acc):
            hs = i // SCG
            g = i % SCG
            yv = y_v[hr * 8 + hs, pl.ds(g * L, L)]
            m = (yv >= lo) & (yv < lo + HC)
            yl = jnp.clip(yv - lo, 0, HC - 1)
            hv = jnp.broadcast_to(hs, (L,)).astype(jnp.int32)
            wv = g * L + lane
            vals = plsc.load_gather(buf, [yl, hv, wv], mask=m)
            return acc - jnp.where(m, vals, 0.0)

        return grp

    nstages = NHR * 2
    cps = {0: start(0), 1: start(1)}
    acc = jnp.zeros((L,), jnp.float32)
    for st in range(nstages):
        cps[st].wait()
        if st + 2 < nstages:
            cps[st + 2] = start(st + 2)
        hr, half = divmod(st, 2)
        acc = lax.fori_loop(0, 8 * SCG, make_grp(hr, half, bufs[st % 2]),
                            acc)

    acc_v[...] = acc
    pltpu.sync_copy(acc_v, out_hbm.at[wid])


@functools.partial(
    pl.kernel,
    out_type=jax.ShapeDtypeStruct((NW, L), jnp.float32),
    mesh=plsc.VectorSubcoreMesh(core_axis_name="c", subcore_axis_name="s"),
    scratch_types=[
        pltpu.VMEM((QH, SCW), jnp.int32),
        pltpu.VMEM((C // 2, 8, TW), jnp.float32),
        pltpu.VMEM((C // 2, 8, TW), jnp.float32),
        pltpu.VMEM((L,), jnp.float32),
        pltpu.SemaphoreType.DMA,
        pltpu.SemaphoreType.DMA,
    ],
    compiler_params=pltpu.CompilerParams(
        use_tc_tiling_on_sc=True, needs_layout_passes=False
    ),
)
def _sc_gather_sum(scores_hbm, y_hbm, out_hbm, y_v, buf0_v, buf1_v, acc_v,
                   sem0, sem1):
    _sc_body(scores_hbm, y_hbm, out_hbm, y_v, buf0_v, buf1_v, acc_v,
             sem0, sem1)


TCH = 224


def _tc_body(s_ref, y_ref, out_ref):
    s = s_ref[0]
    y = y_ref[0]
    ge32 = jnp.broadcast_to((y >= 32)[None], (32, TCH, TCW))
    ge64 = jnp.broadcast_to((y >= 64)[None], (32, TCH, TCW))
    v = jnp.where(ge64, s[64:96], jnp.where(ge32, s[32:64], s[0:32]))
    k = 16
    while k >= 1:
        bit = jnp.broadcast_to((y & k) > 0, (k, TCH, TCW))
        v = jnp.where(bit, v[k:2 * k], v[:k])
        k //= 2
    valid = lax.broadcasted_iota(jnp.int32, (1, TCH, TCW), 2) < TCV

    @pl.when((pl.program_id(0) == 0) & (pl.program_id(1) == 0))
    def _():
        out_ref[0, 0] = 0.0

    out_ref[0, 0] -= jnp.sum(jnp.where(valid, v, 0.0))


def _tc_masked_sum(scores, y):
    return pl.pallas_call(
        _tc_body,
        grid=(B, H // TCH),
        in_specs=[
            pl.BlockSpec((1, C, TCH, TCW), lambda b, h: (b, 0, h, 1)),
            pl.BlockSpec((1, TCH, TCW), lambda b, h: (b, h, 1)),
        ],
        out_specs=pl.BlockSpec(
            (1, 1), lambda b, h: (0, 0), memory_space=pltpu.SMEM
        ),
        out_shape=jax.ShapeDtypeStruct((1, 1), jnp.float32),
    )(scores, y)


def kernel(scores, y):
    sc_partials = _sc_gather_sum(scores, y)
    tc_partials = _tc_masked_sum(scores, y)
    return jnp.sum(sc_partials) + tc_partials[0, 0]


# speedup vs baseline: 1.0244x; 1.0244x over previous
import functools

import jax
import jax.numpy as jnp
from jax import lax
from jax.experimental import pallas as pl
from jax.experimental.pallas import tpu as pltpu
from jax.experimental.pallas import tpu_sc as plsc

B, C, H, W = 8, 96, 224, 224
HWN = H * W
NPIX = B * HWN
NC, NS, L = 2, 16, 16
NW = NC * NS
PPW = NPIX // NW
QH = 56
NHR = QH // 8
TW = 128
SCW = 128
SCG = SCW // L
TCW = 128
TCV = W - SCW


def _sc_body(scores_hbm, y_hbm, out_hbm, y_v, buf_v, acc_v):
    c = lax.axis_index("c")
    s = lax.axis_index("s")
    wid = s * NC + c
    b = wid // 4
    h0 = (wid % 4) * QH

    pltpu.sync_copy(
        y_hbm.at[b, pl.ds(h0, QH), pl.ds(0, SCW)], y_v
    )
    lane = lax.iota(jnp.int32, L)

    def tile_band(hr, acc):
        habs = h0 + hr * 8
        pltpu.sync_copy(
            scores_hbm.at[b, :, pl.ds(habs, 8), pl.ds(0, TW)], buf_v
        )

        def grp(i, acc):
            hs = i // SCG
            g = i % SCG
            yv = y_v[hr * 8 + hs, pl.ds(g * L, L)]
            hv = jnp.broadcast_to(hs, (L,)).astype(jnp.int32)
            wv = g * L + lane
            return acc - plsc.load_gather(buf_v, [yv, hv, wv])

        return lax.fori_loop(0, 8 * SCG, grp, acc)

    acc = lax.fori_loop(0, NHR, tile_band, jnp.zeros((L,), jnp.float32))
    acc_v[...] = acc
    pltpu.sync_copy(acc_v, out_hbm.at[wid])


@functools.partial(
    pl.kernel,
    out_type=jax.ShapeDtypeStruct((NW, L), jnp.float32),
    mesh=plsc.VectorSubcoreMesh(core_axis_name="c", subcore_axis_name="s"),
    scratch_types=[
        pltpu.VMEM((QH, SCW), jnp.int32),
        pltpu.VMEM((C, 8, TW), jnp.float32),
        pltpu.VMEM((L,), jnp.float32),
    ],
    compiler_params=pltpu.CompilerParams(
        use_tc_tiling_on_sc=True, needs_layout_passes=False
    ),
)
def _sc_gather_sum(scores_hbm, y_hbm, out_hbm, y_v, buf_v, acc_v):
    _sc_body(scores_hbm, y_hbm, out_hbm, y_v, buf_v, acc_v)


TCH = 224


def _tc_body(s_ref, y_ref, out_ref):
    s = s_ref[0]
    y = y_ref[0]
    ge32 = jnp.broadcast_to((y >= 32)[None], (32, TCH, TCW))
    ge64 = jnp.broadcast_to((y >= 64)[None], (32, TCH, TCW))
    v = jnp.where(ge64, s[64:96], jnp.where(ge32, s[32:64], s[0:32]))
    k = 16
    while k >= 1:
        bit = jnp.broadcast_to((y & k) > 0, (k, TCH, TCW))
        v = jnp.where(bit, v[k:2 * k], v[:k])
        k //= 2
    valid = lax.broadcasted_iota(jnp.int32, (1, TCH, TCW), 2) < TCV

    @pl.when((pl.program_id(0) == 0) & (pl.program_id(1) == 0))
    def _():
        out_ref[0, 0] = 0.0

    out_ref[0, 0] -= jnp.sum(jnp.where(valid, v, 0.0))


def _tc_masked_sum(scores, y):
    return pl.pallas_call(
        _tc_body,
        grid=(B, H // TCH),
        in_specs=[
            pl.BlockSpec((1, C, TCH, TCW), lambda b, h: (b, 0, h, 1)),
            pl.BlockSpec((1, TCH, TCW), lambda b, h: (b, h, 1)),
        ],
        out_specs=pl.BlockSpec(
            (1, 1), lambda b, h: (0, 0), memory_space=pltpu.SMEM
        ),
        out_shape=jax.ShapeDtypeStruct((1, 1), jnp.float32),
    )(scores, y)


def kernel(scores, y):
    sc_partials = _sc_gather_sum(scores, y)
    tc_partials = _tc_masked_sum(scores, y)
    return jnp.sum(sc_partials) + tc_partials[0, 0]
